# baseline (speedup 1.0000x reference)
import functools
import jax
import jax.numpy as jnp
from jax import lax
from jax.experimental import pallas as pl
from jax.experimental.pallas import tpu as pltpu

_N = 50000
_NP = 51200
_NG = 128
_BLK = 1024
_GRID = _NP // _BLK


def _elu(v):
    return jnp.where(v > 0, v, jnp.exp(jnp.minimum(v, 0.0)) - 1.0)


def _row_spec(w, blk=_BLK):
    return pl.BlockSpec((blk, w), lambda i: (i, 0))


def _full_spec(shape):
    return pl.BlockSpec(shape, lambda i: tuple(0 for _ in shape))


def _dense1_body(x_ref, p0_ref, p1_ref, wr_ref, wn_ref, b_ref,
                 hl_ref, hr_ref, deg_ref):
    p = p0_ref[...] + p1_ref[...]
    deg = p[:, 2:3]
    a = p[:, 0:2] / jnp.maximum(deg, 1.0)
    h = _elu(x_ref[...] @ wr_ref[...] + a @ wn_ref[...] + b_ref[...])
    hl_ref[...] = h[:, :16]
    hr_ref[...] = h[:, 16:]
    deg_ref[...] = deg


def _denseL_body(hl_ref, hr_ref, a0_ref, a1_ref, deg_ref,
                 wr_ref, wn_ref, b_ref, ol_ref, or_ref):
    h = jnp.concatenate([hl_ref[...], hr_ref[...]], axis=1)
    a = jnp.concatenate([a0_ref[...], a1_ref[...]], axis=1)
    a = a / jnp.maximum(deg_ref[...], 1.0)
    o = _elu(h @ wr_ref[...] + a @ wn_ref[...] + b_ref[...])
    half = o.shape[1] // 2
    ol_ref[...] = o[:, :half]
    or_ref[...] = o[:, half:]


def _dense1(xp, p0, p1, W1r, W1n, b1):
    return pl.pallas_call(
        _dense1_body,
        grid=(_GRID,),
        in_specs=[_row_spec(2), _row_spec(16), _row_spec(16),
                  _full_spec((2, 32)), _full_spec((2, 32)), _full_spec((1, 32))],
        out_specs=[_row_spec(16), _row_spec(16), _row_spec(1)],
        out_shape=[jax.ShapeDtypeStruct((_NP, 16), jnp.float32),
                   jax.ShapeDtypeStruct((_NP, 16), jnp.float32),
                   jax.ShapeDtypeStruct((_NP, 1), jnp.float32)],
    )(xp, p0, p1, W1r, W1n, b1.reshape(1, 32))


def _denseL(hl, hr, a0, a1, deg, Wr, Wn, b):
    win = hl.shape[1] * 2
    wout = Wr.shape[1]
    half = wout // 2
    return pl.pallas_call(
        _denseL_body,
        grid=(_GRID,),
        in_specs=[_row_spec(win // 2), _row_spec(win // 2),
                  _row_spec(win // 2), _row_spec(win // 2), _row_spec(1),
                  _full_spec((win, wout)), _full_spec((win, wout)),
                  _full_spec((1, wout))],
        out_specs=[_row_spec(half), _row_spec(half)],
        out_shape=[jax.ShapeDtypeStruct((_NP, half), jnp.float32),
                   jax.ShapeDtypeStruct((_NP, half), jnp.float32)],
    )(hl, hr, a0, a1, deg, Wr, Wn, b.reshape(1, wout))


_SB = 512


def _segmax_body(bat_ref, hl_ref, hr_ref, out_ref):
    i = pl.program_id(0)

    @pl.when(i == 0)
    def _init():
        out_ref[...] = jnp.full(out_ref.shape, -jnp.inf, jnp.float32)

    g = bat_ref[...]
    h = jnp.concatenate([hl_ref[...], hr_ref[...]], axis=1)
    for gg in range(_NG):
        cand = jnp.where(g == gg, h, -jnp.inf)
        red = jnp.max(cand, axis=0, keepdims=True)
        out_ref[gg:gg + 1, :] = jnp.maximum(out_ref[gg:gg + 1, :], red)


def _segmax(batp, hl, hr):
    return pl.pallas_call(
        _segmax_body,
        grid=(_NP // _SB,),
        in_specs=[_row_spec(1, _SB), _row_spec(32, _SB), _row_spec(32, _SB)],
        out_specs=pl.BlockSpec((_NG, 64), lambda i: (0, 0)),
        out_shape=jax.ShapeDtypeStruct((_NG, 64), jnp.float32),
    )(batp, hl, hr)


def _head_body(g_ref, w1_ref, b1_ref, w2_ref, b2_ref, w3_ref, b3_ref, out_ref):
    g = g_ref[...]
    g = _elu(g @ w1_ref[...] + b1_ref[...])
    g = _elu(g @ w2_ref[...] + b2_ref[...])
    v = g @ w3_ref[...] + b3_ref[...]
    m = jnp.max(v, axis=1, keepdims=True)
    lse = m + jnp.log(jnp.sum(jnp.exp(v - m), axis=1, keepdims=True))
    out_ref[...] = v - lse


def _head(g, fc1w, fc1b, fc2w, fc2b, fc3w, fc3b):
    return pl.pallas_call(
        _head_body,
        grid=(1,),
        in_specs=[_full_spec((_NG, 64)),
                  _full_spec((64, 64)), _full_spec((1, 64)),
                  _full_spec((64, 32)), _full_spec((1, 32)),
                  _full_spec((32, 2)), _full_spec((1, 2))],
        out_specs=_full_spec((_NG, 2)),
        out_shape=jax.ShapeDtypeStruct((_NG, 2), jnp.float32),
    )(g, fc1w, fc1b.reshape(1, 64), fc2w, fc2b.reshape(1, 32),
      fc3w, fc3b.reshape(1, 2))


def kernel(x, edge_index, batch, W1r, W1n, b1, W2r, W2n, b2,
           L0r, L0n, bl0, L1r, L1n, bl1, L2r, L2n, bl2,
           fc1w, fc1b, fc2w, fc2b, fc3w, fc3b):
    pad = _NP - _N
    xp = jnp.pad(x, ((0, pad), (0, 0)))
    batp = jnp.pad(batch, (0, pad), constant_values=_NG)[:, None]
    src = edge_index[0]
    dst = edge_index[1]

    x_aug = jnp.concatenate(
        [xp, jnp.ones((_NP, 1), jnp.float32)], axis=1)
    aggA = jax.ops.segment_sum(x_aug[src], dst, num_segments=_NP)
    p0 = jnp.pad(aggA, ((0, 0), (0, 13)))
    p1 = jnp.zeros_like(p0)

    hl, hr, deg = _dense1(xp, p0, p1, W1r, W1n, b1)

    a0 = jax.ops.segment_sum(hl[src], dst, num_segments=_NP)
    a1 = jax.ops.segment_sum(hr[src], dst, num_segments=_NP)
    hl, hr = _denseL(hl, hr, a0, a1, deg, W2r, W2n, b2)

    for Wr, Wn, b in ((L0r, L0n, bl0), (L1r, L1n, bl1), (L2r, L2n, bl2)):
        a0 = jax.ops.segment_sum(hl[src], dst, num_segments=_NP)
        a1 = jax.ops.segment_sum(hr[src], dst, num_segments=_NP)
        hl, hr = _denseL(hl, hr, a0, a1, deg, Wr, Wn, b)

    g = _segmax(batp, hl, hr)
    return _head(g, fc1w, fc1b, fc2w, fc2b, fc3w, fc3b)


# speedup vs baseline: 5.9125x; 5.9125x over previous
import functools
import jax
import jax.numpy as jnp
from jax import lax
from jax.experimental import pallas as pl
from jax.experimental.pallas import tpu as pltpu
from jax.experimental.pallas import tpu_sc as plsc

_N = 50000
_NP = 51200
_NG = 128
_BLK = 1024
_GRID = _NP // _BLK
_TPN = _NP // 16
_CK = 128


_G = 8


def _sc_spmm(tab, src_idx, dst_idx, zeros, w2, nchunks):
    ngroups = nchunks // _G
    mesh = plsc.VectorSubcoreMesh(core_axis_name="c", subcore_axis_name="s")

    @functools.partial(
        pl.kernel, mesh=mesh,
        compiler_params=pltpu.CompilerParams(use_tc_tiling_on_sc=False),
        out_type=jax.ShapeDtypeStruct((2, _NP, w2), jnp.float32),
        scratch_types=[
            pltpu.VMEM_SHARED((_NP, w2), jnp.float32),
            pltpu.VMEM((_G, _CK), jnp.int32),
            pltpu.VMEM((_G, _CK), jnp.int32),
            pltpu.VMEM((_CK, w2), jnp.float32),
            pltpu.VMEM((_CK, w2), jnp.float32),
            pltpu.SemaphoreType.DMA,
            pltpu.SemaphoreType.DMA,
        ])
    def k(tab_hbm, src_hbm, dst_hbm, z_hbm, out_hbm,
          acc, src_v, dst_v, r0, r1, s0, s1):
        c = lax.axis_index("c")
        s = lax.axis_index("s")
        pltpu.sync_copy(z_hbm, acc.at[pl.ds(s * _TPN, _TPN)])
        plsc.subcore_barrier()

        bufs = ((r0, s0), (r1, s1))

        def group(g, carry):
            pltpu.sync_copy(src_hbm.at[c, s, pl.ds(g * _G, _G)], src_v)
            pltpu.sync_copy(dst_hbm.at[c, s, pl.ds(g * _G, _G)], dst_v)
            for b in range(2):
                pltpu.make_async_copy(tab_hbm.at[src_v.at[b]], bufs[b][0],
                                      bufs[b][1]).start()
            for j in range(_G):
                buf, sem = bufs[j % 2]
                pltpu.make_async_copy(tab_hbm.at[src_v.at[j]], buf, sem).wait()
                pltpu.sync_copy(buf, acc.at[dst_v.at[j]], add=True)
                if j + 2 < _G:
                    pltpu.make_async_copy(tab_hbm.at[src_v.at[j + 2]], buf,
                                          sem).start()
            return carry

        lax.fori_loop(0, ngroups, group, 0)
        plsc.subcore_barrier()
        pltpu.sync_copy(acc.at[pl.ds(s * _TPN, _TPN)],
                        out_hbm.at[c, pl.ds(s * _TPN, _TPN)])

    return k(tab, src_idx, dst_idx, zeros)


def _elu(v):
    return jnp.where(v > 0, v, jnp.exp(jnp.minimum(v, 0.0)) - 1.0)


def _row_spec(w, blk=_BLK):
    return pl.BlockSpec((blk, w), lambda i: (i, 0))


def _full_spec(shape):
    return pl.BlockSpec(shape, lambda i: tuple(0 for _ in shape))


def _dense1_body(x_ref, p0_ref, p1_ref, wr_ref, wn_ref, b_ref,
                 hl_ref, hr_ref, deg_ref):
    p = p0_ref[...] + p1_ref[...]
    deg = p[:, 2:3]
    a = p[:, 0:2] / jnp.maximum(deg, 1.0)
    h = _elu(x_ref[...] @ wr_ref[...] + a @ wn_ref[...] + b_ref[...])
    hl_ref[...] = h[:, :16]
    hr_ref[...] = h[:, 16:]
    deg_ref[...] = deg


def _denseL_body(hl_ref, hr_ref, a0_ref, a1_ref, deg_ref,
                 wr_ref, wn_ref, b_ref, ol_ref, or_ref):
    h = jnp.concatenate([hl_ref[...], hr_ref[...]], axis=1)
    a = jnp.concatenate([a0_ref[...], a1_ref[...]], axis=1)
    a = a / jnp.maximum(deg_ref[...], 1.0)
    o = _elu(h @ wr_ref[...] + a @ wn_ref[...] + b_ref[...])
    half = o.shape[1] // 2
    ol_ref[...] = o[:, :half]
    or_ref[...] = o[:, half:]


def _dense1(xp, p0, p1, W1r, W1n, b1):
    return pl.pallas_call(
        _dense1_body,
        grid=(_GRID,),
        in_specs=[_row_spec(2), _row_spec(16), _row_spec(16),
                  _full_spec((2, 32)), _full_spec((2, 32)), _full_spec((1, 32))],
        out_specs=[_row_spec(16), _row_spec(16), _row_spec(1)],
        out_shape=[jax.ShapeDtypeStruct((_NP, 16), jnp.float32),
                   jax.ShapeDtypeStruct((_NP, 16), jnp.float32),
                   jax.ShapeDtypeStruct((_NP, 1), jnp.float32)],
    )(xp, p0, p1, W1r, W1n, b1.reshape(1, 32))


def _denseL(hl, hr, a0, a1, deg, Wr, Wn, b):
    win = hl.shape[1] * 2
    wout = Wr.shape[1]
    half = wout // 2
    return pl.pallas_call(
        _denseL_body,
        grid=(_GRID,),
        in_specs=[_row_spec(win // 2), _row_spec(win // 2),
                  _row_spec(win // 2), _row_spec(win // 2), _row_spec(1),
                  _full_spec((win, wout)), _full_spec((win, wout)),
                  _full_spec((1, wout))],
        out_specs=[_row_spec(half), _row_spec(half)],
        out_shape=[jax.ShapeDtypeStruct((_NP, half), jnp.float32),
                   jax.ShapeDtypeStruct((_NP, half), jnp.float32)],
    )(hl, hr, a0, a1, deg, Wr, Wn, b.reshape(1, wout))


_SB = 512


def _segmax_body(bat_ref, hl_ref, hr_ref, out_ref):
    i = pl.program_id(0)

    @pl.when(i == 0)
    def _init():
        out_ref[...] = jnp.full(out_ref.shape, -jnp.inf, jnp.float32)

    g = bat_ref[...]
    h = jnp.concatenate([hl_ref[...], hr_ref[...]], axis=1)
    for gg in range(_NG):
        cand = jnp.where(g == gg, h, -jnp.inf)
        red = jnp.max(cand, axis=0, keepdims=True)
        out_ref[gg:gg + 1, :] = jnp.maximum(out_ref[gg:gg + 1, :], red)


def _segmax(batp, hl, hr):
    return pl.pallas_call(
        _segmax_body,
        grid=(_NP // _SB,),
        in_specs=[_row_spec(1, _SB), _row_spec(32, _SB), _row_spec(32, _SB)],
        out_specs=pl.BlockSpec((_NG, 64), lambda i: (0, 0)),
        out_shape=jax.ShapeDtypeStruct((_NG, 64), jnp.float32),
    )(batp, hl, hr)


def _head_body(g_ref, w1_ref, b1_ref, w2_ref, b2_ref, w3_ref, b3_ref, out_ref):
    g = g_ref[...]
    g = _elu(g @ w1_ref[...] + b1_ref[...])
    g = _elu(g @ w2_ref[...] + b2_ref[...])
    v = g @ w3_ref[...] + b3_ref[...]
    m = jnp.max(v, axis=1, keepdims=True)
    lse = m + jnp.log(jnp.sum(jnp.exp(v - m), axis=1, keepdims=True))
    out_ref[...] = v - lse


def _head(g, fc1w, fc1b, fc2w, fc2b, fc3w, fc3b):
    return pl.pallas_call(
        _head_body,
        grid=(1,),
        in_specs=[_full_spec((_NG, 64)),
                  _full_spec((64, 64)), _full_spec((1, 64)),
                  _full_spec((64, 32)), _full_spec((1, 32)),
                  _full_spec((32, 2)), _full_spec((1, 2))],
        out_specs=_full_spec((_NG, 2)),
        out_shape=jax.ShapeDtypeStruct((_NG, 2), jnp.float32),
    )(g, fc1w, fc1b.reshape(1, 64), fc2w, fc2b.reshape(1, 32),
      fc3w, fc3b.reshape(1, 2))


def kernel(x, edge_index, batch, W1r, W1n, b1, W2r, W2n, b2,
           L0r, L0n, bl0, L1r, L1n, bl1, L2r, L2n, bl2,
           fc1w, fc1b, fc2w, fc2b, fc3w, fc3b):
    pad = _NP - _N
    xp = jnp.pad(x, ((0, pad), (0, 0)))
    batp = jnp.pad(batch, (0, pad), constant_values=_NG)[:, None]
    src = edge_index[0]
    dst = edge_index[1]
    E = src.shape[0]

    nca = 200
    pada = 2 * 16 * nca * _CK - E
    srcA = jnp.concatenate([src, jnp.full((pada,), _N, jnp.int32)]) \
        .reshape(2, 16, nca, _CK)
    dstA = jnp.concatenate([dst, jnp.full((pada,), _N, jnp.int32)]) \
        .reshape(2, 16, nca, _CK)
    ncb = 400
    padb = 16 * ncb * _CK - E
    srcB = jnp.concatenate([src.reshape(16, E // 16),
                            jnp.full((16, padb // 16), _N, jnp.int32)],
                           axis=1).reshape(16, ncb, _CK)
    dstB = jnp.concatenate([dst.reshape(16, E // 16),
                            jnp.full((16, padb // 16), _N, jnp.int32)],
                           axis=1).reshape(16, ncb, _CK)
    srcB2 = jnp.stack([srcB, srcB + _NP])
    dstB2 = jnp.stack([dstB, dstB])
    zeros16 = jnp.zeros((_TPN, 16), jnp.float32)
    zeros32 = jnp.zeros((_TPN, 32), jnp.float32)

    x_aug = jnp.concatenate(
        [xp, jnp.ones((_NP, 1), jnp.float32),
         jnp.zeros((_NP, 13), jnp.float32)], axis=1)
    aggA = _sc_spmm(x_aug, srcA, dstA, zeros16, 16, nca)
    hl, hr, deg = _dense1(xp, aggA[0], aggA[1], W1r, W1n, b1)

    agg = _sc_spmm(jnp.concatenate([hl, hr], axis=0), srcB2, dstB2,
                   zeros16, 16, ncb)
    hl, hr = _denseL(hl, hr, agg[0], agg[1], deg, W2r, W2n, b2)

    for Wr, Wn, b in ((L0r, L0n, bl0), (L1r, L1n, bl1), (L2r, L2n, bl2)):
        agg = _sc_spmm(jnp.concatenate([hl, hr], axis=0), srcB2, dstB2,
                       zeros32, 32, ncb)
        hl, hr = _denseL(hl, hr, agg[0], agg[1], deg, Wr, Wn, b)

    g = _segmax(batp, hl, hr)
    return _head(g, fc1w, fc1b, fc2w, fc2b, fc3w, fc3b)


# speedup vs baseline: 6.6364x; 1.1224x over previous
import functools
import jax
import jax.numpy as jnp
from jax import lax
from jax.experimental import pallas as pl
from jax.experimental.pallas import tpu as pltpu
from jax.experimental.pallas import tpu_sc as plsc

_N = 50000
_NP = 51200
_NG = 128
_BLK = 1024
_GRID = _NP // _BLK
_TPN = _NP // 16
_CK = 128
_D = 4
_K = 3


def _sc_spmm(tab, cidx, zeros, w, G, ngroups):
    Q = ngroups // 2
    mesh = plsc.VectorSubcoreMesh(core_axis_name="c", subcore_axis_name="s")

    @functools.partial(
        pl.kernel, mesh=mesh,
        compiler_params=pltpu.CompilerParams(use_tc_tiling_on_sc=False),
        out_type=jax.ShapeDtypeStruct((2, _NP, w), jnp.float32),
        scratch_types=[
            pltpu.VMEM_SHARED((_NP, w), jnp.float32),
            pltpu.VMEM((2 * G, _CK), jnp.int32),
            pltpu.VMEM((2 * G, _CK), jnp.int32),
            pltpu.VMEM((_D, _CK, w), jnp.float32),
            pltpu.SemaphoreType.DMA,
            pltpu.SemaphoreType.DMA,
            pltpu.SemaphoreType.DMA,
            pltpu.SemaphoreType.DMA,
            pltpu.SemaphoreType.DMA,
            pltpu.SemaphoreType.DMA,
            pltpu.SemaphoreType.DMA,
            pltpu.SemaphoreType.DMA,
            pltpu.SemaphoreType.DMA,
        ])
    def k(tab_hbm, cidx_hbm, z_hbm, out_hbm, acc, idx0, idx1, rows,
          isem, g0, g1, g2, g3, s0, s1, s2, s3):
        c = lax.axis_index("c")
        s = lax.axis_index("s")
        gsem = (g0, g1, g2, g3)
        ssem = (s0, s1, s2, s3)
        pltpu.sync_copy(z_hbm, acc.at[pl.ds(s * _TPN, _TPN)])
        plsc.subcore_barrier()

        def group(g, idxp, idxq, p, first, last):
            for j in range(G):
                u = j + _K
                if not (last and u >= G):
                    if u == G and not last:
                        pltpu.make_async_copy(cidx_hbm.at[c, s, g + 1], idxq,
                                              isem).wait()
                    srow = idxp.at[u] if u < G else idxq.at[u - G]
                    b_u = (p * G + u) % _D
                    if not (first and u < _D):
                        pltpu.make_async_copy(tab_hbm.at[pl.ds(0, _CK)],
                                              rows.at[b_u], ssem[b_u]).wait()
                    pltpu.async_copy(tab_hbm.at[srow], rows.at[b_u], gsem[b_u])
                if j == 0 and not last:
                    pltpu.async_copy(cidx_hbm.at[c, s, g + 1], idxq, isem)
                b_t = (p * G + j) % _D
                pltpu.make_async_copy(tab_hbm.at[idxp.at[j]], rows.at[b_t],
                                      gsem[b_t]).wait()
                pltpu.async_copy(rows.at[b_t], acc.at[idxp.at[G + j]],
                                 ssem[b_t], add=True)

        pltpu.sync_copy(cidx_hbm.at[c, s, 0], idx0)
        for u in range(_K):
            pltpu.async_copy(tab_hbm.at[idx0.at[u]], rows.at[u % _D],
                             gsem[u % _D])

        group(0, idx0, idx1, 0, True, False)
        group(1, idx1, idx0, 1, False, False)

        def pair(q, carry):
            group(2 * q, idx0, idx1, 0, False, False)
            group(2 * q + 1, idx1, idx0, 1, False, False)
            return carry

        lax.fori_loop(1, Q - 1, pair, 0)

        group(2 * Q - 2, idx0, idx1, 0, False, False)
        group(2 * Q - 1, idx1, idx0, 1, False, True)

        for b in range(_D):
            pltpu.make_async_copy(tab_hbm.at[pl.ds(0, _CK)], rows.at[b],
                                  ssem[b]).wait()
        plsc.subcore_barrier()
        pltpu.sync_copy(acc.at[pl.ds(s * _TPN, _TPN)],
                        out_hbm.at[c, pl.ds(s * _TPN, _TPN)])

    return k(tab, cidx, zeros)


def _elu(v):
    return jnp.where(v > 0, v, jnp.exp(jnp.minimum(v, 0.0)) - 1.0)


def _row_spec(w, blk=_BLK):
    return pl.BlockSpec((blk, w), lambda i: (i, 0))


def _full_spec(shape):
    return pl.BlockSpec(shape, lambda i: tuple(0 for _ in shape))


def _dense1_body(x_ref, p0_ref, p1_ref, wr_ref, wn_ref, b_ref,
                 hl_ref, hr_ref, deg_ref):
    p = p0_ref[...] + p1_ref[...]
    deg = p[:, 2:3]
    a = p[:, 0:2] / jnp.maximum(deg, 1.0)
    h = _elu(x_ref[...] @ wr_ref[...] + a @ wn_ref[...] + b_ref[...])
    hl_ref[...] = h[:, :16]
    hr_ref[...] = h[:, 16:]
    deg_ref[...] = deg


def _dense2_body(hl_ref, hr_ref, p0_ref, p1_ref, deg_ref,
                 wr_ref, wn_ref, b_ref, ol_ref, or_ref):
    h = jnp.concatenate([hl_ref[...], hr_ref[...]], axis=1)
    a = (p0_ref[...] + p1_ref[...]) / jnp.maximum(deg_ref[...], 1.0)
    o = _elu(h @ wr_ref[...] + a @ wn_ref[...] + b_ref[...])
    half = o.shape[1] // 2
    ol_ref[...] = o[:, :half]
    or_ref[...] = o[:, half:]


def _denseL_body(hl_ref, hr_ref, a0_ref, a1_ref, deg_ref,
                 wr_ref, wn_ref, b_ref, ol_ref, or_ref):
    h = jnp.concatenate([hl_ref[...], hr_ref[...]], axis=1)
    a = jnp.concatenate([a0_ref[...], a1_ref[...]], axis=1)
    a = a / jnp.maximum(deg_ref[...], 1.0)
    o = _elu(h @ wr_ref[...] + a @ wn_ref[...] + b_ref[...])
    half = o.shape[1] // 2
    ol_ref[...] = o[:, :half]
    or_ref[...] = o[:, half:]


def _dense1(xp, p0, p1, W1r, W1n, b1):
    return pl.pallas_call(
        _dense1_body,
        grid=(_GRID,),
        in_specs=[_row_spec(2), _row_spec(16), _row_spec(16),
                  _full_spec((2, 32)), _full_spec((2, 32)), _full_spec((1, 32))],
        out_specs=[_row_spec(16), _row_spec(16), _row_spec(1)],
        out_shape=[jax.ShapeDtypeStruct((_NP, 16), jnp.float32),
                   jax.ShapeDtypeStruct((_NP, 16), jnp.float32),
                   jax.ShapeDtypeStruct((_NP, 1), jnp.float32)],
    )(xp, p0, p1, W1r, W1n, b1.reshape(1, 32))


def _dense2(hl, hr, p0, p1, deg, Wr, Wn, b):
    win = hl.shape[1] * 2
    wout = Wr.shape[1]
    half = wout // 2
    return pl.pallas_call(
        _dense2_body,
        grid=(_GRID,),
        in_specs=[_row_spec(win // 2), _row_spec(win // 2),
                  _row_spec(win), _row_spec(win), _row_spec(1),
                  _full_spec((win, wout)), _full_spec((win, wout)),
                  _full_spec((1, wout))],
        out_specs=[_row_spec(half), _row_spec(half)],
        out_shape=[jax.ShapeDtypeStruct((_NP, half), jnp.float32),
                   jax.ShapeDtypeStruct((_NP, half), jnp.float32)],
    )(hl, hr, p0, p1, deg, Wr, Wn, b.reshape(1, wout))


def _denseL(hl, hr, a0, a1, deg, Wr, Wn, b):
    win = hl.shape[1] * 2
    wout = Wr.shape[1]
    half = wout // 2
    return pl.pallas_call(
        _denseL_body,
        grid=(_GRID,),
        in_specs=[_row_spec(win // 2), _row_spec(win // 2),
                  _row_spec(win // 2), _row_spec(win // 2), _row_spec(1),
                  _full_spec((win, wout)), _full_spec((win, wout)),
                  _full_spec((1, wout))],
        out_specs=[_row_spec(half), _row_spec(half)],
        out_shape=[jax.ShapeDtypeStruct((_NP, half), jnp.float32),
                   jax.ShapeDtypeStruct((_NP, half), jnp.float32)],
    )(hl, hr, a0, a1, deg, Wr, Wn, b.reshape(1, wout))


_SB = 512


def _segmax_body(bat_ref, hl_ref, hr_ref, out_ref):
    i = pl.program_id(0)

    @pl.when(i == 0)
    def _init():
        out_ref[...] = jnp.full(out_ref.shape, -jnp.inf, jnp.float32)

    g = bat_ref[...]
    h = jnp.concatenate([hl_ref[...], hr_ref[...]], axis=1)
    for gg in range(_NG):
        cand = jnp.where(g == gg, h, -jnp.inf)
        red = jnp.max(cand, axis=0, keepdims=True)
        out_ref[gg:gg + 1, :] = jnp.maximum(out_ref[gg:gg + 1, :], red)


def _segmax(batp, hl, hr):
    return pl.pallas_call(
        _segmax_body,
        grid=(_NP // _SB,),
        in_specs=[_row_spec(1, _SB), _row_spec(32, _SB), _row_spec(32, _SB)],
        out_specs=pl.BlockSpec((_NG, 64), lambda i: (0, 0)),
        out_shape=jax.ShapeDtypeStruct((_NG, 64), jnp.float32),
    )(batp, hl, hr)


def _head_body(g_ref, w1_ref, b1_ref, w2_ref, b2_ref, w3_ref, b3_ref, out_ref):
    g = g_ref[...]
    g = _elu(g @ w1_ref[...] + b1_ref[...])
    g = _elu(g @ w2_ref[...] + b2_ref[...])
    v = g @ w3_ref[...] + b3_ref[...]
    m = jnp.max(v, axis=1, keepdims=True)
    lse = m + jnp.log(jnp.sum(jnp.exp(v - m), axis=1, keepdims=True))
    out_ref[...] = v - lse


def _head(g, fc1w, fc1b, fc2w, fc2b, fc3w, fc3b):
    return pl.pallas_call(
        _head_body,
        grid=(1,),
        in_specs=[_full_spec((_NG, 64)),
                  _full_spec((64, 64)), _full_spec((1, 64)),
                  _full_spec((64, 32)), _full_spec((1, 32)),
                  _full_spec((32, 2)), _full_spec((1, 2))],
        out_specs=_full_spec((_NG, 2)),
        out_shape=jax.ShapeDtypeStruct((_NG, 2), jnp.float32),
    )(g, fc1w, fc1b.reshape(1, 64), fc2w, fc2b.reshape(1, 32),
      fc3w, fc3b.reshape(1, 2))


_GA = 10
_NGA = 20
_GB = 8
_NGB = 50


def kernel(x, edge_index, batch, W1r, W1n, b1, W2r, W2n, b2,
           L0r, L0n, bl0, L1r, L1n, bl1, L2r, L2n, bl2,
           fc1w, fc1b, fc2w, fc2b, fc3w, fc3b):
    pad = _NP - _N
    xp = jnp.pad(x, ((0, pad), (0, 0)))
    batp = jnp.pad(batch, (0, pad), constant_values=_NG)[:, None]
    src = edge_index[0]
    dst = edge_index[1]
    E = src.shape[0]

    nca = _NGA * _GA
    pada = 2 * 16 * nca * _CK - E
    srcA = jnp.concatenate([src, jnp.full((pada,), _N, jnp.int32)]) \
        .reshape(2, 16, _NGA, _GA, _CK)
    dstA = jnp.concatenate([dst, jnp.full((pada,), _N, jnp.int32)]) \
        .reshape(2, 16, _NGA, _GA, _CK)
    cidxA = jnp.concatenate([srcA, dstA], axis=3)
    ncb = _NGB * _GB
    padb = 16 * ncb * _CK - E
    srcB = jnp.concatenate([src.reshape(16, E // 16),
                            jnp.full((16, padb // 16), _N, jnp.int32)],
                           axis=1).reshape(16, _NGB, _GB, _CK)
    dstB = jnp.concatenate([dst.reshape(16, E // 16),
                            jnp.full((16, padb // 16), _N, jnp.int32)],
                           axis=1).reshape(16, _NGB, _GB, _CK)
    srcB2 = jnp.stack([srcB, srcB + _NP])
    dstB2 = jnp.stack([dstB, dstB])
    cidxB = jnp.concatenate([srcB2, dstB2], axis=3)
    zeros16 = jnp.zeros((_TPN, 16), jnp.float32)
    zeros32 = jnp.zeros((_TPN, 32), jnp.float32)

    x_aug = jnp.concatenate(
        [xp, jnp.ones((_NP, 1), jnp.float32),
         jnp.zeros((_NP, 13), jnp.float32)], axis=1)
    aggA = _sc_spmm(x_aug, cidxA, zeros16, 16, _GA, _NGA)
    hl, hr, deg = _dense1(xp, aggA[0], aggA[1], W1r, W1n, b1)

    agg2 = _sc_spmm(jnp.concatenate([hl, hr], axis=1), cidxA, zeros32,
                    32, _GA, _NGA)
    hl, hr = _dense2(hl, hr, agg2[0], agg2[1], deg, W2r, W2n, b2)

    for Wr, Wn, b in ((L0r, L0n, bl0), (L1r, L1n, bl1), (L2r, L2n, bl2)):
        agg = _sc_spmm(jnp.concatenate([hl, hr], axis=0), cidxB, zeros32,
                       32, _GB, _NGB)
        hl, hr = _denseL(hl, hr, agg[0], agg[1], deg, Wr, Wn, b)

    g = _segmax(batp, hl, hr)
    return _head(g, fc1w, fc1b, fc2w, fc2b, fc3w, fc3b)


# speedup vs baseline: 7.6879x; 1.1584x over previous
import functools
import jax
import jax.numpy as jnp
from jax import lax
from jax.experimental import pallas as pl
from jax.experimental.pallas import tpu as pltpu
from jax.experimental.pallas import tpu_sc as plsc

_N = 50000
_NP = 51200
_NG = 128
_BLK = 1024
_GRID = _NP // _BLK
_TPN = _NP // 16
_CK = 128
_D = 4
_K = 3


def _sc_spmm(tab, srcg, dstg, zeros, w, G, ngroups):
    Q = ngroups // 2
    edge_split = srcg.ndim == 5
    mesh = plsc.VectorSubcoreMesh(core_axis_name="c", subcore_axis_name="s")

    @functools.partial(
        pl.kernel, mesh=mesh,
        compiler_params=pltpu.CompilerParams(use_tc_tiling_on_sc=False),
        out_type=jax.ShapeDtypeStruct((2, _NP, w), jnp.float32),
        scratch_types=[
            pltpu.VMEM_SHARED((_NP, w), jnp.float32),
            pltpu.VMEM((2, G, _CK), jnp.int32),
            pltpu.VMEM((2, G, _CK), jnp.int32),
            pltpu.VMEM((_D, _CK, w), jnp.float32),
            pltpu.SemaphoreType.DMA,
            pltpu.SemaphoreType.DMA,
            pltpu.SemaphoreType.DMA,
            pltpu.SemaphoreType.DMA,
            pltpu.SemaphoreType.DMA,
            pltpu.SemaphoreType.DMA,
            pltpu.SemaphoreType.DMA,
            pltpu.SemaphoreType.DMA,
            pltpu.SemaphoreType.DMA,
        ])
    def k(tab_hbm, src_hbm, dst_hbm, z_hbm, out_hbm, acc, idx0, idx1, rows,
          isem, g0, g1, g2, g3, s0, s1, s2, s3):
        c = lax.axis_index("c")
        s = lax.axis_index("s")
        gsem = (g0, g1, g2, g3)
        ssem = (s0, s1, s2, s3)

        def src_at(g):
            return src_hbm.at[c, s, g] if edge_split else src_hbm.at[s, g]

        def dst_at(g):
            return dst_hbm.at[c, s, g] if edge_split else dst_hbm.at[s, g]

        def tab_at(row_ref):
            return (tab_hbm.at[row_ref] if edge_split
                    else tab_hbm.at[c].at[row_ref])

        pltpu.sync_copy(z_hbm, acc.at[pl.ds(s * _TPN, _TPN)])
        plsc.subcore_barrier()

        def group(g, idxp, idxq, p, first, last):
            for j in range(G):
                u = j + _K
                if not (last and u >= G):
                    if u == G:
                        pltpu.make_async_copy(src_at(g + 1), idxq.at[0],
                                              isem).wait()
                        pltpu.make_async_copy(dst_at(g + 1), idxq.at[1],
                                              isem).wait()
                    srow = idxp.at[0, u] if u < G else idxq.at[0, u - G]
                    b_u = (p * G + u) % _D
                    if not (first and u < _D):
                        pltpu.make_async_copy(z_hbm.at[pl.ds(0, _CK)],
                                              rows.at[b_u], ssem[b_u]).wait()
                    pltpu.async_copy(tab_at(srow), rows.at[b_u], gsem[b_u])
                if j == 0 and not last:
                    pltpu.async_copy(src_at(g + 1), idxq.at[0], isem)
                    pltpu.async_copy(dst_at(g + 1), idxq.at[1], isem)
                b_t = (p * G + j) % _D
                pltpu.make_async_copy(tab_at(idxp.at[0, j]), rows.at[b_t],
                                      gsem[b_t]).wait()
                pltpu.async_copy(rows.at[b_t], acc.at[idxp.at[1, j]],
                                 ssem[b_t], add=True)

        pltpu.sync_copy(src_at(0), idx0.at[0])
        pltpu.sync_copy(dst_at(0), idx0.at[1])
        for u in range(_K):
            pltpu.async_copy(tab_at(idx0.at[0, u]), rows.at[u % _D],
                             gsem[u % _D])

        group(0, idx0, idx1, 0, True, False)
        group(1, idx1, idx0, 1, False, False)

        def pair(q, carry):
            group(2 * q, idx0, idx1, 0, False, False)
            group(2 * q + 1, idx1, idx0, 1, False, False)
            return carry

        lax.fori_loop(1, Q - 1, pair, 0)

        group(2 * Q - 2, idx0, idx1, 0, False, False)
        group(2 * Q - 1, idx1, idx0, 1, False, True)

        for b in range(_D):
            pltpu.make_async_copy(z_hbm.at[pl.ds(0, _CK)], rows.at[b],
                                  ssem[b]).wait()
        plsc.subcore_barrier()
        pltpu.sync_copy(acc.at[pl.ds(s * _TPN, _TPN)],
                        out_hbm.at[c, pl.ds(s * _TPN, _TPN)])

    return k(tab, srcg, dstg, zeros)


def _elu(v):
    return jnp.where(v > 0, v, jnp.exp(jnp.minimum(v, 0.0)) - 1.0)


def _row_spec(w, blk=_BLK):
    return pl.BlockSpec((blk, w), lambda i: (i, 0))


def _stk_spec(w, blk=_BLK):
    return pl.BlockSpec((2, blk, w), lambda i: (0, i, 0))


def _full_spec(shape):
    return pl.BlockSpec(shape, lambda i: tuple(0 for _ in shape))


def _dense1_body(x_ref, p_ref, wr_ref, wn_ref, b_ref, h_ref, deg_ref):
    p = p_ref[0] + p_ref[1]
    deg = p[:, 2:3]
    a = p[:, 0:2] / jnp.maximum(deg, 1.0)
    h_ref[...] = _elu(x_ref[...] @ wr_ref[...] + a @ wn_ref[...] + b_ref[...])
    deg_ref[...] = deg


def _dense2_body(h_ref, p_ref, deg_ref, wr_ref, wn_ref, b_ref, out_ref):
    a = (p_ref[0] + p_ref[1]) / jnp.maximum(deg_ref[...], 1.0)
    o = _elu(h_ref[...] @ wr_ref[...] + a @ wn_ref[...] + b_ref[...])
    half = o.shape[1] // 2
    out_ref[0] = o[:, :half]
    out_ref[1] = o[:, half:]


def _denseL_body(s_ref, p_ref, deg_ref, wr_ref, wn_ref, b_ref, out_ref):
    h = jnp.concatenate([s_ref[0], s_ref[1]], axis=1)
    a = jnp.concatenate([p_ref[0], p_ref[1]], axis=1)
    a = a / jnp.maximum(deg_ref[...], 1.0)
    o = _elu(h @ wr_ref[...] + a @ wn_ref[...] + b_ref[...])
    half = o.shape[1] // 2
    out_ref[0] = o[:, :half]
    out_ref[1] = o[:, half:]


def _dense1(xp, agg, W1r, W1n, b1):
    return pl.pallas_call(
        _dense1_body,
        grid=(_GRID,),
        in_specs=[_row_spec(2), _stk_spec(16),
                  _full_spec((2, 32)), _full_spec((2, 32)), _full_spec((1, 32))],
        out_specs=[_row_spec(32), _row_spec(1)],
        out_shape=[jax.ShapeDtypeStruct((_NP, 32), jnp.float32),
                   jax.ShapeDtypeStruct((_NP, 1), jnp.float32)],
    )(xp, agg, W1r, W1n, b1.reshape(1, 32))


def _dense2(h, agg, deg, Wr, Wn, b):
    wout = Wr.shape[1]
    return pl.pallas_call(
        _dense2_body,
        grid=(_GRID,),
        in_specs=[_row_spec(32), _stk_spec(32), _row_spec(1),
                  _full_spec((32, wout)), _full_spec((32, wout)),
                  _full_spec((1, wout))],
        out_specs=_stk_spec(wout // 2),
        out_shape=jax.ShapeDtypeStruct((2, _NP, wout // 2), jnp.float32),
    )(h, agg, deg, Wr, Wn, b.reshape(1, wout))


def _denseL(s, agg, deg, Wr, Wn, b):
    wout = Wr.shape[1]
    return pl.pallas_call(
        _denseL_body,
        grid=(_GRID,),
        in_specs=[_stk_spec(32), _stk_spec(32), _row_spec(1),
                  _full_spec((64, wout)), _full_spec((64, wout)),
                  _full_spec((1, wout))],
        out_specs=_stk_spec(wout // 2),
        out_shape=jax.ShapeDtypeStruct((2, _NP, wout // 2), jnp.float32),
    )(s, agg, deg, Wr, Wn, b.reshape(1, wout))


_SB = 512


def _segmax_body(bat_ref, s_ref, out_ref):
    i = pl.program_id(0)

    @pl.when(i == 0)
    def _init():
        out_ref[...] = jnp.full(out_ref.shape, -jnp.inf, jnp.float32)

    g = bat_ref[...]
    h = jnp.concatenate([s_ref[0], s_ref[1]], axis=1)
    for gg in range(_NG):
        cand = jnp.where(g == gg, h, -jnp.inf)
        red = jnp.max(cand, axis=0, keepdims=True)
        out_ref[gg:gg + 1, :] = jnp.maximum(out_ref[gg:gg + 1, :], red)


def _segmax(batp, s):
    return pl.pallas_call(
        _segmax_body,
        grid=(_NP // _SB,),
        in_specs=[_row_spec(1, _SB), _stk_spec(32, _SB)],
        out_specs=pl.BlockSpec((_NG, 64), lambda i: (0, 0)),
        out_shape=jax.ShapeDtypeStruct((_NG, 64), jnp.float32),
    )(batp, s)


def _head_body(g_ref, w1_ref, b1_ref, w2_ref, b2_ref, w3_ref, b3_ref, out_ref):
    g = g_ref[...]
    g = _elu(g @ w1_ref[...] + b1_ref[...])
    g = _elu(g @ w2_ref[...] + b2_ref[...])
    v = g @ w3_ref[...] + b3_ref[...]
    m = jnp.max(v, axis=1, keepdims=True)
    lse = m + jnp.log(jnp.sum(jnp.exp(v - m), axis=1, keepdims=True))
    out_ref[...] = v - lse


def _head(g, fc1w, fc1b, fc2w, fc2b, fc3w, fc3b):
    return pl.pallas_call(
        _head_body,
        grid=(1,),
        in_specs=[_full_spec((_NG, 64)),
                  _full_spec((64, 64)), _full_spec((1, 64)),
                  _full_spec((64, 32)), _full_spec((1, 32)),
                  _full_spec((32, 2)), _full_spec((1, 2))],
        out_specs=_full_spec((_NG, 2)),
        out_shape=jax.ShapeDtypeStruct((_NG, 2), jnp.float32),
    )(g, fc1w, fc1b.reshape(1, 64), fc2w, fc2b.reshape(1, 32),
      fc3w, fc3b.reshape(1, 2))


_GA = 10
_NGA = 20
_GB = 8
_NGB = 50


def kernel(x, edge_index, batch, W1r, W1n, b1, W2r, W2n, b2,
           L0r, L0n, bl0, L1r, L1n, bl1, L2r, L2n, bl2,
           fc1w, fc1b, fc2w, fc2b, fc3w, fc3b):
    pad = _NP - _N
    xp = jnp.pad(x, ((0, pad), (0, 0)))
    batp = jnp.pad(batch, (0, pad), constant_values=_NG)[:, None]
    src = edge_index[0]
    dst = edge_index[1]
    E = src.shape[0]

    nca = _NGA * _GA
    pada = 2 * 16 * nca * _CK - E
    srcA = jnp.concatenate([src, jnp.full((pada,), _N, jnp.int32)]) \
        .reshape(2, 16, _NGA, _GA, _CK)
    dstA = jnp.concatenate([dst, jnp.full((pada,), _N, jnp.int32)]) \
        .reshape(2, 16, _NGA, _GA, _CK)
    ncb = _NGB * _GB
    padb = 16 * ncb * _CK - E
    srcB = jnp.concatenate([src.reshape(16, E // 16),
                            jnp.full((16, padb // 16), _N, jnp.int32)],
                           axis=1).reshape(16, _NGB, _GB, _CK)
    dstB = jnp.concatenate([dst.reshape(16, E // 16),
                            jnp.full((16, padb // 16), _N, jnp.int32)],
                           axis=1).reshape(16, _NGB, _GB, _CK)
    zeros16 = jnp.zeros((_TPN, 16), jnp.float32)
    zeros32 = jnp.zeros((_TPN, 32), jnp.float32)

    x_aug = jnp.concatenate(
        [xp, jnp.ones((_NP, 1), jnp.float32),
         jnp.zeros((_NP, 13), jnp.float32)], axis=1)
    aggA = _sc_spmm(x_aug, srcA, dstA, zeros16, 16, _GA, _NGA)
    h, deg = _dense1(xp, aggA, W1r, W1n, b1)

    agg2 = _sc_spmm(h, srcA, dstA, zeros32, 32, _GA, _NGA)
    s = _dense2(h, agg2, deg, W2r, W2n, b2)

    for Wr, Wn, b in ((L0r, L0n, bl0), (L1r, L1n, bl1), (L2r, L2n, bl2)):
        agg = _sc_spmm(s, srcB, dstB, zeros32, 32, _GB, _NGB)
        s = _denseL(s, agg, deg, Wr, Wn, b)

    g = _segmax(batp, s)
    return _head(g, fc1w, fc1b, fc2w, fc2b, fc3w, fc3b)
